# baseline (speedup 1.0000x reference)
import functools

import jax
import jax.numpy as jnp
from jax import lax
from jax.experimental import pallas as pl
from jax.experimental.pallas import tpu as pltpu
from jax.experimental.pallas import tpu_sc as plsc

B, Q, HM, WM = 4, 200, 128, 128
T, D = 1024, 768
C1 = 81
TOP_K = 16
EPS = 1e-06
GH = 32
NSLOT = B * TOP_K


def _scores_topk_body(logits_ref, scores_ref, tks_ref, tki_ref, tkc_ref, fidx_ref):
    l = logits_ref[...]
    m = jnp.max(l, axis=-1, keepdims=True)
    p = jnp.exp(l - m)
    s = jnp.sum(p, axis=-1, keepdims=True)
    probs = p / s
    fg = probs[:, :, : C1 - 1]
    sc3 = jnp.max(fg, axis=-1, keepdims=True)
    cio = lax.broadcasted_iota(jnp.int32, (B, Q, C1 - 1), 2)
    cls3 = jnp.min(jnp.where(fg == sc3, cio, C1), axis=-1, keepdims=True)

    ii = lax.broadcasted_iota(jnp.int32, (Q, Q), 0)
    jj = lax.broadcasted_iota(jnp.int32, (Q, Q), 1)
    ro = lax.broadcasted_iota(jnp.int32, (Q, TOP_K), 1)
    qio = lax.broadcasted_iota(jnp.int32, (Q, TOP_K), 0)

    for b in range(B):
        sc = sc3[b]
        sc_row = sc.T
        scores_ref[pl.ds(b, 1), :] = sc_row

        beats = (sc_row > sc) | ((sc_row == sc) & (jj < ii))
        rank = jnp.sum(beats.astype(jnp.int32), axis=-1, keepdims=True)

        onehot = rank == ro
        tki_row = jnp.sum(jnp.where(onehot, qio, 0), axis=0, keepdims=True)
        tks_ref[pl.ds(b, 1), :] = jnp.sum(jnp.where(onehot, sc, 0.0), axis=0, keepdims=True)
        tki_ref[pl.ds(b, 1), :] = tki_row
        tkc_ref[pl.ds(b, 1), :] = jnp.sum(jnp.where(onehot, cls3[b], 0), axis=0, keepdims=True)
        so = lax.broadcasted_iota(jnp.int32, (1, 2 * GH), 1)
        off = ((so >> 1) << 2) + 1 + (so & 1)
        gq = HM * (tki_row.T + b * Q)
        fidx_ref[pl.ds(b * TOP_K, TOP_K), :] = gq + off


def _scores_topk(class_logits):
    return pl.pallas_call(
        _scores_topk_body,
        out_shape=(
            jax.ShapeDtypeStruct((B, Q), jnp.float32),
            jax.ShapeDtypeStruct((B, TOP_K), jnp.float32),
            jax.ShapeDtypeStruct((B, TOP_K), jnp.int32),
            jax.ShapeDtypeStruct((B, TOP_K), jnp.int32),
            jax.ShapeDtypeStruct((NSLOT, 2 * GH), jnp.int32),
        ),
    )(class_logits)


def _sc_body(ridx_hbm, rows_hbm, out_hbm, idx_v, rows_v, sem):
    info = plsc.get_sparse_core_info()
    wid = lax.axis_index("s") * info.num_cores + lax.axis_index("c")

    pltpu.sync_copy(ridx_hbm.at[pl.ds(wid * 2, 2)], idx_v)
    c0 = pltpu.async_copy(rows_hbm.at[idx_v.at[0]], rows_v.at[0], sem)
    c1 = pltpu.async_copy(rows_hbm.at[idx_v.at[1]], rows_v.at[1], sem)
    c0.wait()
    c1.wait()
    pltpu.sync_copy(rows_v, out_hbm.at[pl.ds(wid * 2, 2)])


def _sc_gather_rows(ridx, soft_masks):
    rows = soft_masks.reshape(B * Q * HM, WM)
    mesh = plsc.VectorSubcoreMesh(core_axis_name="c", subcore_axis_name="s")
    k = functools.partial(
        pl.kernel,
        out_type=jax.ShapeDtypeStruct((NSLOT, 2 * GH, WM), jnp.float32),
        mesh=mesh,
        scratch_types=[
            pltpu.VMEM((2, 2 * GH), jnp.int32),
            pltpu.VMEM((2, 2 * GH, WM), jnp.float32),
            pltpu.SemaphoreType.DMA,
        ],
    )(_sc_body)
    return k(ridx, rows)


def _pool_body(rs_ref, v_ref, out_ref, w_scratch):
    rs = rs_ref[0]
    r3 = rs.reshape(TOP_K * GH, 2, WM)
    rowsum = r3[:, 0, :] + r3[:, 1, :]
    rT = lax.broadcasted_iota(jnp.int32, (WM, GH), 0)
    cT = lax.broadcasted_iota(jnp.int32, (WM, GH), 1)
    selT = ((rT == 4 * cT + 1) | (rT == 4 * cT + 2)).astype(jnp.float32)
    resized_all = 0.25 * jnp.dot(rowsum, selT, precision=lax.Precision.HIGHEST,
                                 preferred_element_type=jnp.float32)

    for k in range(TOP_K):
        w_scratch[:, k, :] = resized_all[k * GH:(k + 1) * GH, :]

    num = jnp.zeros((TOP_K, D), jnp.float32)
    den = jnp.zeros((TOP_K, 1), jnp.float32)
    for g in range(GH):
        sg = w_scratch[g]
        vg = v_ref[0, pl.ds(g * GH, GH), :]
        num = num + jnp.dot(sg, vg, preferred_element_type=jnp.float32)
        den = den + jnp.sum(sg, axis=1, keepdims=True)
    out_ref[0] = num / (den + EPS)


def _pool(rs, visual_features):
    return pl.pallas_call(
        _pool_body,
        grid=(B,),
        in_specs=[
            pl.BlockSpec((1, TOP_K * 2 * GH, WM), lambda b: (b, 0, 0)),
            pl.BlockSpec((1, T, D), lambda b: (b, 0, 0)),
        ],
        out_specs=pl.BlockSpec((1, TOP_K, D), lambda b: (b, 0, 0)),
        out_shape=jax.ShapeDtypeStruct((B, TOP_K, D), jnp.float32),
        scratch_shapes=[pltpu.VMEM((GH, TOP_K, GH), jnp.float32)],
    )(rs, visual_features)


@functools.partial(jax.jit)
def kernel(soft_masks, visual_features, class_logits):
    scores, topk_scores, topk_indices, topk_class_ids, ridx = _scores_topk(class_logits)
    rs = _sc_gather_rows(ridx, soft_masks).reshape(B, TOP_K * 2 * GH, WM)
    pooled = _pool(rs, visual_features)
    return pooled, topk_scores, topk_indices, topk_class_ids, scores


# speedup vs baseline: 1.3550x; 1.3550x over previous
import functools

import jax
import jax.numpy as jnp
from jax import lax
from jax.experimental import pallas as pl
from jax.experimental.pallas import tpu as pltpu

B, Q, HM, WM = 4, 200, 128, 128
T, D = 1024, 768
C1 = 81
TOP_K = 16
EPS = 1e-06
GH = 32


def _scores_topk_body(logits_ref, scores_ref, tks_ref, tki_ref, tkc_ref):
    l = logits_ref[...]
    m = jnp.max(l, axis=-1, keepdims=True)
    p = jnp.exp(l - m)
    s = jnp.sum(p, axis=-1, keepdims=True)
    probs = p / s
    fg = probs[:, :, : C1 - 1]
    sc3 = jnp.max(fg, axis=-1, keepdims=True)
    cio = lax.broadcasted_iota(jnp.int32, (B, Q, C1 - 1), 2)
    cls3 = jnp.min(jnp.where(fg == sc3, cio, C1), axis=-1, keepdims=True)

    ii = lax.broadcasted_iota(jnp.int32, (Q, Q), 0)
    jj = lax.broadcasted_iota(jnp.int32, (Q, Q), 1)
    ro = lax.broadcasted_iota(jnp.int32, (Q, TOP_K), 1)
    qio = lax.broadcasted_iota(jnp.int32, (Q, TOP_K), 0)

    for b in range(B):
        sc = sc3[b]
        sc_row = sc.T
        scores_ref[pl.ds(b, 1), :] = sc_row

        beats = (sc_row > sc) | ((sc_row == sc) & (jj < ii))
        rank = jnp.sum(beats.astype(jnp.int32), axis=-1, keepdims=True)

        onehot = rank == ro
        tks_ref[pl.ds(b, 1), :] = jnp.sum(jnp.where(onehot, sc, 0.0), axis=0, keepdims=True)
        tki_ref[pl.ds(b, 1), :] = jnp.sum(jnp.where(onehot, qio, 0), axis=0, keepdims=True)
        tkc_ref[pl.ds(b, 1), :] = jnp.sum(jnp.where(onehot, cls3[b], 0), axis=0, keepdims=True)


def _resize_2x2(m):
    m4 = m.reshape(GH, 4, WM)
    rowsum = m4[:, 1, :] + m4[:, 2, :]
    rt4 = rowsum.T.reshape(GH, 4, GH)
    return (0.25 * (rt4[:, 1, :] + rt4[:, 2, :])).T


NC = 4
GPC = GH // NC


def _pool_body(idx_ref, *refs):
    mask_refs = refs[:TOP_K]
    v_ref = refs[TOP_K]
    out_ref = refs[TOP_K + 1]
    w_scratch = refs[TOP_K + 2]
    acc_ref = refs[TOP_K + 3]
    c = pl.program_id(1)

    @pl.when(c == 0)
    def _():
        for k in range(TOP_K):
            w_scratch[:, k, :] = _resize_2x2(mask_refs[k][0, 0])

    num = jnp.zeros((TOP_K, D), jnp.float32)
    for gl in range(GPC):
        sg = w_scratch[c * GPC + gl]
        vg = v_ref[0, pl.ds(gl * GH, GH), :]
        num = num + jnp.dot(sg, vg, preferred_element_type=jnp.float32)
    acc_ref[...] = jnp.where(c == 0, num, acc_ref[...] + num)

    @pl.when(c == NC - 1)
    def _():
        den = jnp.sum(jnp.sum(w_scratch[...], axis=2), axis=0, keepdims=True)
        out_ref[0] = acc_ref[...] / (den.T + EPS)


def _scores_topk(class_logits):
    return pl.pallas_call(
        _scores_topk_body,
        out_shape=(
            jax.ShapeDtypeStruct((B, Q), jnp.float32),
            jax.ShapeDtypeStruct((B, TOP_K), jnp.float32),
            jax.ShapeDtypeStruct((B, TOP_K), jnp.int32),
            jax.ShapeDtypeStruct((B, TOP_K), jnp.int32),
        ),
    )(class_logits)


def _pool(topk_indices, soft_masks, visual_features):
    mask_specs = [
        pl.BlockSpec((1, 1, HM, WM), lambda b, c, idx, k=k: (b, idx[b, k], 0, 0))
        for k in range(TOP_K)
    ]
    grid_spec = pltpu.PrefetchScalarGridSpec(
        num_scalar_prefetch=1,
        grid=(B, NC),
        in_specs=mask_specs
        + [pl.BlockSpec((1, T // NC, D), lambda b, c, idx: (b, c, 0))],
        out_specs=pl.BlockSpec((1, TOP_K, D), lambda b, c, idx: (b, 0, 0)),
        scratch_shapes=[
            pltpu.VMEM((GH, TOP_K, GH), jnp.float32),
            pltpu.VMEM((TOP_K, D), jnp.float32),
        ],
    )
    return pl.pallas_call(
        _pool_body,
        grid_spec=grid_spec,
        out_shape=jax.ShapeDtypeStruct((B, TOP_K, D), jnp.float32),
    )(topk_indices, *([soft_masks] * TOP_K), visual_features)


@functools.partial(jax.jit)
def kernel(soft_masks, visual_features, class_logits):
    scores, topk_scores, topk_indices, topk_class_ids = _scores_topk(class_logits)
    pooled = _pool(topk_indices, soft_masks, visual_features)
    return pooled, topk_scores, topk_indices, topk_class_ids, scores


# speedup vs baseline: 2.2264x; 1.6430x over previous
import functools

import jax
import jax.numpy as jnp
from jax import lax
from jax.experimental import pallas as pl
from jax.experimental.pallas import tpu as pltpu

B, Q, HM, WM = 4, 200, 128, 128
T, D = 1024, 768
C1 = 81
TOP_K = 16
EPS = 1e-06
GH = 32


def _scores_topk_body(logits_ref, scores_ref, tks_ref, tki_ref, tkc_ref):
    l = logits_ref[...]
    m = jnp.max(l, axis=-1, keepdims=True)
    p = jnp.exp(l - m)
    s = jnp.sum(p, axis=-1, keepdims=True)
    probs = p / s
    fg = probs[:, :, : C1 - 1]
    sc3 = jnp.max(fg, axis=-1, keepdims=True)
    cio = lax.broadcasted_iota(jnp.int32, (B, Q, C1 - 1), 2)
    cls3 = jnp.min(jnp.where(fg == sc3, cio, C1), axis=-1, keepdims=True)

    ii = lax.broadcasted_iota(jnp.int32, (Q, Q), 0)
    jj = lax.broadcasted_iota(jnp.int32, (Q, Q), 1)
    ro = lax.broadcasted_iota(jnp.int32, (Q, TOP_K), 1)
    qio = lax.broadcasted_iota(jnp.int32, (Q, TOP_K), 0)

    for b in range(B):
        sc = sc3[b]
        sc_row = sc.T
        scores_ref[pl.ds(b, 1), :] = sc_row

        beats = (sc_row > sc) | ((sc_row == sc) & (jj < ii))
        rank = jnp.sum(beats.astype(jnp.int32), axis=-1, keepdims=True)

        onehot = rank == ro
        tks_ref[pl.ds(b, 1), :] = jnp.sum(jnp.where(onehot, sc, 0.0), axis=0, keepdims=True)
        tki_ref[pl.ds(b, 1), :] = jnp.sum(jnp.where(onehot, qio, 0), axis=0, keepdims=True)
        tkc_ref[pl.ds(b, 1), :] = jnp.sum(jnp.where(onehot, cls3[b], 0), axis=0, keepdims=True)


def _resize_2x2(m):
    m4 = m.reshape(GH, 4, WM)
    rowsum = m4[:, 1, :] + m4[:, 2, :]
    rt4 = rowsum.T.reshape(GH, 4, GH)
    return (0.25 * (rt4[:, 1, :] + rt4[:, 2, :])).T


def _pool_body(idx_ref, *refs):
    mask_refs = refs[:TOP_K]
    v_ref = refs[TOP_K]
    out_ref = refs[TOP_K + 1]
    w_scratch = refs[TOP_K + 2]

    for k in range(TOP_K):
        resized = _resize_2x2(mask_refs[k][0, 0])
        w_scratch[:, k, :] = resized

    num = jnp.zeros((TOP_K, D), jnp.float32)
    den = jnp.zeros((TOP_K, 1), jnp.float32)
    for g in range(GH):
        sg = w_scratch[g]
        vg = v_ref[0, pl.ds(g * GH, GH), :]
        num = num + jnp.dot(sg, vg, preferred_element_type=jnp.float32)
        den = den + jnp.sum(sg, axis=1, keepdims=True)
    out_ref[0] = num / (den + EPS)


def _scores_topk(class_logits):
    return pl.pallas_call(
        _scores_topk_body,
        out_shape=(
            jax.ShapeDtypeStruct((B, Q), jnp.float32),
            jax.ShapeDtypeStruct((B, TOP_K), jnp.float32),
            jax.ShapeDtypeStruct((B, TOP_K), jnp.int32),
            jax.ShapeDtypeStruct((B, TOP_K), jnp.int32),
        ),
    )(class_logits)


def _pool(topk_indices, soft_masks, visual_features):
    mask_specs = [
        pl.BlockSpec((1, 1, HM, WM), lambda b, idx, k=k: (b, idx[b, k], 0, 0))
        for k in range(TOP_K)
    ]
    grid_spec = pltpu.PrefetchScalarGridSpec(
        num_scalar_prefetch=1,
        grid=(B,),
        in_specs=mask_specs + [pl.BlockSpec((1, T, D), lambda b, idx: (b, 0, 0))],
        out_specs=pl.BlockSpec((1, TOP_K, D), lambda b, idx: (b, 0, 0)),
        scratch_shapes=[
            pltpu.VMEM((GH, TOP_K, GH), jnp.float32),
        ],
    )
    return pl.pallas_call(
        _pool_body,
        grid_spec=grid_spec,
        out_shape=jax.ShapeDtypeStruct((B, TOP_K, D), jnp.float32),
    )(topk_indices, *([soft_masks] * TOP_K), visual_features)


@functools.partial(jax.jit)
def kernel(soft_masks, visual_features, class_logits):
    scores, topk_scores, topk_indices, topk_class_ids = _scores_topk(class_logits)
    pooled = _pool(topk_indices, soft_masks, visual_features)
    return pooled, topk_scores, topk_indices, topk_class_ids, scores
